# baseline (speedup 1.0000x reference)
import functools

import jax
import jax.numpy as jnp
from jax import lax
from jax.experimental import pallas as pl
from jax.experimental.pallas import tpu as pltpu
from jax.experimental.pallas import tpu_sc as plsc

_VOCAB = 1000000
_DIM = 64
_BATCH = 16384

_UNROLL = 16
_NSEM = 8


def _make_gather(V, D, B):
  info = plsc.get_sparse_core_info()
  NC, NS = info.num_cores, info.num_subcores
  NW = NC * NS
  b_per_w = B // NW
  mesh = plsc.VectorSubcoreMesh(core_axis_name="c", subcore_axis_name="s")

  @functools.partial(
      pl.kernel,
      mesh=mesh,
      out_type=jax.ShapeDtypeStruct((B, D), jnp.float32),
      scratch_types=[
          pltpu.VMEM((b_per_w,), jnp.int32),
          pltpu.VMEM((b_per_w, D), jnp.float32),
          [pltpu.SemaphoreType.DMA] * _NSEM,
      ],
  )
  def k(table_hbm, idx_hbm, out_hbm, idx_v, rows_v, sems):
    wid = lax.axis_index("s") * NC + lax.axis_index("c")
    base = wid * b_per_w

    pltpu.sync_copy(idx_hbm.at[pl.ds(base, b_per_w)], idx_v)

    def body(it, carry):
      j0 = it * _UNROLL
      v = idx_v[pl.ds(j0, _UNROLL)]
      for t in range(_UNROLL):
        pltpu.async_copy(
            table_hbm.at[v[t]], rows_v.at[j0 + t], sems[t % _NSEM]
        )
      return carry

    lax.fori_loop(0, b_per_w // _UNROLL, body, 0)
    for s in range(_NSEM):
      pltpu.make_async_copy(
          table_hbm.at[pl.ds(0, b_per_w // _NSEM)],
          rows_v.at[pl.ds(0, b_per_w // _NSEM)],
          sems[s],
      ).wait()
    pltpu.sync_copy(rows_v, out_hbm.at[pl.ds(base, b_per_w)])

  return k, NW


_gather, _NW = _make_gather(_VOCAB, _DIM, _BATCH)


@jax.jit
def kernel(indices, table):
  return _gather(table, indices.astype(jnp.int32))


# speedup vs baseline: 1.4991x; 1.4991x over previous
import functools

import jax
import jax.numpy as jnp
from jax import lax
from jax.experimental import pallas as pl
from jax.experimental.pallas import tpu as pltpu
from jax.experimental.pallas import tpu_sc as plsc

_VOCAB = 1000000
_DIM = 64
_BATCH = 16384

_TAIL_START = 999936
_CHUNK_COLS = 512
_NCHUNK = _TAIL_START // _CHUNK_COLS
_NSLOT = 64


def _make_gather(V, D, B):
  info = plsc.get_sparse_core_info()
  NC, NS = info.num_cores, info.num_subcores
  NW = NC * NS
  mesh = plsc.VectorSubcoreMesh(core_axis_name="c", subcore_axis_name="s")

  @functools.partial(
      pl.kernel,
      mesh=mesh,
      out_type=jax.ShapeDtypeStruct((B, D), jnp.float32),
      scratch_types=[
          pltpu.VMEM((B,), jnp.int32),
          pltpu.VMEM((B + 32,), jnp.int32),
          pltpu.VMEM((B + 32,), jnp.int32),
          pltpu.VMEM((32,), jnp.int32),
          pltpu.VMEM((32,), jnp.int32),
          pltpu.VMEM((D, _CHUNK_COLS), jnp.float32),
          pltpu.VMEM((_NSLOT, D), jnp.float32),
          pltpu.VMEM((V - _TAIL_START, D), jnp.float32),
          pltpu.SemaphoreType.DMA,
      ],
      compiler_params=pltpu.CompilerParams(needs_layout_passes=False),
  )
  def k(tab_t, tail, idx_hbm, out_hbm, idx_v, hidx_v, hb_v, sub_v, subb_v,
        chunk_v, stage_v, tail_v, osem):
    w = lax.axis_index("s") * NC + lax.axis_index("c")
    c0 = (w * _NCHUNK) // NW
    c1 = ((w + 1) * _NCHUNK) // NW
    lo = c0 * _CHUNK_COLS
    hi = jnp.where(w == NW - 1, V, c1 * _CHUNK_COLS)

    pltpu.sync_copy(idx_hbm, idx_v)
    pltpu.sync_copy(tail, tail_v)

    iota = lax.iota(jnp.int32, 16)

    def scan_body(g, cnt):
      v = idx_v[pl.ds(g * 16, 16)]
      m = (v >= lo) & (v < hi)
      bvec = iota + g * 16
      plsc.store_compressed(hidx_v.at[pl.ds(cnt, 16)], v, mask=m)
      plsc.store_compressed(hb_v.at[pl.ds(cnt, 16)], bvec, mask=m)
      return cnt + plsc.all_reduce_population_count(m)[0]

    nh = lax.fori_loop(0, B // 16, scan_body, jnp.int32(0))
    hidx_v[pl.ds(nh, 16)] = jnp.full((16,), -1, jnp.int32)
    ngrp = (nh + 15) // 16

    def emit(col_fetch, bscal, cnt_out):
      slot = jnp.bitwise_and(cnt_out, _NSLOT - 1)

      @pl.when(cnt_out >= _NSLOT)
      def _():
        pltpu.make_async_copy(stage_v.at[0], out_hbm.at[0], osem).wait()

      for g3 in range(D // 16):
        stage_v[slot, pl.ds(g3 * 16, 16)] = col_fetch(g3)
      pltpu.async_copy(stage_v.at[slot], out_hbm.at[bscal], osem)
      return cnt_out + 1

    def chunk_body(c, cnt_out):
      col0 = pl.multiple_of(c * _CHUNK_COLS, _CHUNK_COLS)
      pltpu.sync_copy(tab_t.at[:, pl.ds(col0, _CHUNK_COLS)], chunk_v)

      def grp_body(g2, cnt_out):
        hv = hidx_v[pl.ds(g2 * 16, 16)]
        bv = hb_v[pl.ds(g2 * 16, 16)]
        m2 = (hv >= col0) & (hv < col0 + _CHUNK_COLS)
        plsc.store_compressed(sub_v.at[pl.ds(0, 16)], hv, mask=m2)
        plsc.store_compressed(subb_v.at[pl.ds(0, 16)], bv, mask=m2)
        mcount = plsc.all_reduce_population_count(m2)[0]

        def match_body(e, cnt_out):
          col = sub_v[pl.ds(e, 16)][0] - col0
          bscal = subb_v[pl.ds(e, 16)][0]
          colv = jnp.full((16,), col, jnp.int32)
          return emit(
              lambda g3: plsc.load_gather(chunk_v, [iota + g3 * 16, colv]),
              bscal, cnt_out)

        return lax.fori_loop(0, mcount, match_body, cnt_out)

      return lax.fori_loop(0, ngrp, grp_body, cnt_out)

    cnt_out = lax.fori_loop(c0, c1, chunk_body, jnp.int32(0))

    def tail_grp_body(g2, cnt_out):
      hv = hidx_v[pl.ds(g2 * 16, 16)]
      bv = hb_v[pl.ds(g2 * 16, 16)]
      m3 = hv >= _TAIL_START
      plsc.store_compressed(sub_v.at[pl.ds(0, 16)], hv, mask=m3)
      plsc.store_compressed(subb_v.at[pl.ds(0, 16)], bv, mask=m3)
      mcount = plsc.all_reduce_population_count(m3)[0]

      def match_body(e, cnt_out):
        r = sub_v[pl.ds(e, 16)][0] - _TAIL_START
        bscal = subb_v[pl.ds(e, 16)][0]
        rv = jnp.full((16,), r, jnp.int32)
        return emit(
            lambda g3: plsc.load_gather(tail_v, [rv, iota + g3 * 16]),
            bscal, cnt_out)

      return lax.fori_loop(0, mcount, match_body, cnt_out)

    ntail_grp = jnp.where(w == NW - 1, ngrp, 0)
    cnt_out = lax.fori_loop(0, ntail_grp, tail_grp_body, cnt_out)

    def drain_body(_, x):
      pltpu.make_async_copy(stage_v.at[0], out_hbm.at[0], osem).wait()
      return x

    lax.fori_loop(0, jnp.minimum(cnt_out, _NSLOT), drain_body, jnp.int32(0))

  return k


_scan_gather = _make_gather(_VOCAB, _DIM, _BATCH)


@jax.jit
def kernel(indices, table):
  table_t = table.T
  tail = table[_TAIL_START:]
  return _scan_gather(table_t, tail, indices.astype(jnp.int32))
